# Initial kernel scaffold; baseline (speedup 1.0000x reference)
import jax
import jax.numpy as jnp
from jax.experimental import pallas as pl


def kernel(x_seq, edge_index_seq, W_self, W_neigh, b_diff, W_mix, b_mix, W_dt, b_dt, A_diag, B, C, Dv, W_res, b_res):
    raise NotImplementedError("write your pallas kernel here")


# speedup vs baseline: 4.6641x; 4.6641x over previous
import jax
import jax.numpy as jnp
from jax import lax
from jax.experimental import pallas as pl
from jax.experimental.pallas import tpu as pltpu
from jax.experimental.pallas import tpu_sc as plsc

L, N, E = 4, 10000, 320000
IN_C, HID, STATE = 128, 256, 16

NC, NS = 2, 16
HC = IN_C // NC
CHUNK = 128
EPW = ((E + NS * CHUNK - 1) // (NS * CHUNK)) * CHUNK
NCHUNK = EPW // CHUNK
EPAD = EPW * NS
NPAD = ((N + NS * 8 - 1) // (NS * 8)) * (NS * 8)
ROWS = NPAD // NS


def _sc_agg_body(xcols, srcq, dstq, zagg, zdeg, ones_src,
                 agg_out, deg_out,
                 src_v, dst_v, rows_v, ones_v, agg_sh, deg_sh, sem):
    cid = lax.axis_index("c")
    sid = lax.axis_index("s")

    pltpu.sync_copy(ones_src, ones_v)

    for l in range(L):
        pltpu.sync_copy(zagg.at[pl.ds(sid * ROWS, ROWS)],
                        agg_sh.at[pl.ds(sid * ROWS, ROWS)])
        pltpu.sync_copy(zdeg.at[pl.ds(sid * ROWS, ROWS)],
                        deg_sh.at[pl.ds(sid * ROWS, ROWS)])
        plsc.subcore_barrier()

        pltpu.sync_copy(srcq.at[cid, l, sid], src_v)
        pltpu.sync_copy(dstq.at[l, sid], dst_v)

        def chunk_body(j, carry):
            pltpu.async_copy(xcols.at[src_v.at[j]], rows_v, sem).wait()
            pltpu.sync_copy(rows_v, agg_sh.at[dst_v.at[j]], add=True)
            pltpu.sync_copy(ones_v, deg_sh.at[dst_v.at[j]], add=True)
            return carry

        lax.fori_loop(0, NCHUNK, chunk_body, 0)
        plsc.subcore_barrier()

        pltpu.sync_copy(agg_sh.at[pl.ds(sid * ROWS, ROWS)],
                        agg_out.at[l, cid, pl.ds(sid * ROWS, ROWS)])
        pltpu.sync_copy(deg_sh.at[pl.ds(sid * ROWS, ROWS)],
                        deg_out.at[l, cid, pl.ds(sid * ROWS, ROWS)])
        plsc.subcore_barrier()


def _sc_agg_call(xcols, srcq, dstq):
    zagg = jnp.zeros((NPAD, HC), jnp.float32)
    zdeg = jnp.zeros((NPAD, 16), jnp.float32)
    ones_src = jnp.ones((CHUNK, 16), jnp.float32)
    return pl.kernel(
        _sc_agg_body,
        out_type=(
            jax.ShapeDtypeStruct((L, NC, NPAD, HC), jnp.float32),
            jax.ShapeDtypeStruct((L, NC, NPAD, 16), jnp.float32),
        ),
        mesh=plsc.VectorSubcoreMesh(core_axis_name="c", subcore_axis_name="s"),
        compiler_params=pltpu.CompilerParams(use_tc_tiling_on_sc=False),
        scratch_types=[
            pltpu.VMEM((NCHUNK, CHUNK), jnp.int32),
            pltpu.VMEM((NCHUNK, CHUNK), jnp.int32),
            pltpu.VMEM((CHUNK, HC), jnp.float32),
            pltpu.VMEM((CHUNK, 16), jnp.float32),
            pltpu.VMEM_SHARED((NPAD, HC), jnp.float32),
            pltpu.VMEM_SHARED((NPAD, 16), jnp.float32),
            pltpu.SemaphoreType.DMA,
        ],
    )(xcols, srcq, dstq, zagg, zdeg, ones_src)


NB = 1000


def _tc_dense_body(x_ref, agg_ref, deg_ref, wsr_ref, wn_ref, bdiff_ref,
                   wm1_ref, wm2_ref, bmix_ref, wdt_ref, bdt_ref, a_ref,
                   b_ref, c_ref, dv_ref, bres_ref, out_ref):
    s_a = jax.nn.softplus(a_ref[...])
    z_prev = None
    u_prev = None
    for l in range(L):
        x = x_ref[l]
        agg = jnp.concatenate([agg_ref[l, 0], agg_ref[l, 1]], axis=-1)
        deg = deg_ref[l, 0, :, 0:1]
        aggn = agg / jnp.maximum(deg, 1.0)
        xsr = jnp.dot(x, wsr_ref[...], preferred_element_type=jnp.float32)
        z = (xsr[:, :HID]
             + jnp.dot(aggn, wn_ref[...], preferred_element_type=jnp.float32)
             + bdiff_ref[...])
        if l == 0:
            h = z
        else:
            gate = jax.nn.sigmoid(
                jnp.dot(z, wm1_ref[...], preferred_element_type=jnp.float32)
                + jnp.dot(z_prev, wm2_ref[...], preferred_element_type=jnp.float32)
                + bmix_ref[...])
            h = gate * z + (1.0 - gate) * z_prev
        dt = jax.nn.softplus(
            jnp.sum(h * wdt_ref[...], axis=1, keepdims=True) + bdt_ref[...])
        decay = jnp.exp(-dt * s_a)
        h_b = jnp.dot(h, b_ref[...], preferred_element_type=jnp.float32)
        if l == 0:
            u = h_b * dt
        else:
            u = u_prev * decay + h_b * dt
        y = (jnp.dot(u, c_ref[...], preferred_element_type=jnp.float32)
             + h * dv_ref[...])
        y_hat = y + xsr[:, HID:] + bres_ref[...]
        out_ref[l] = y_hat * jax.nn.sigmoid(y_hat)
        z_prev = z
        u_prev = u


def _tc_dense_call(x_seq, agg, deg, w_sr, w_neigh, b_diff, wm1, wm2, b_mix,
                   wdt_row, b_dt, a_diag, b_mat, c_mat, dv, b_res):
    grid = (N // NB,)
    full = lambda shape: pl.BlockSpec(shape, lambda i: (0,) * len(shape))
    return pl.pallas_call(
        _tc_dense_body,
        grid=grid,
        in_specs=[
            pl.BlockSpec((L, NB, IN_C), lambda i: (0, i, 0)),
            pl.BlockSpec((L, NC, NB, HC), lambda i: (0, 0, i, 0)),
            pl.BlockSpec((L, NC, NB, 16), lambda i: (0, 0, i, 0)),
            full((IN_C, 2 * HID)),
            full((IN_C, HID)),
            full((1, HID)),
            full((HID, HID)),
            full((HID, HID)),
            full((1, HID)),
            full((1, HID)),
            full((1, 1)),
            full((1, STATE)),
            full((HID, STATE)),
            full((STATE, HID)),
            full((1, HID)),
            full((1, HID)),
        ],
        out_specs=pl.BlockSpec((L, NB, HID), lambda i: (0, i, 0)),
        out_shape=jax.ShapeDtypeStruct((L, N, HID), jnp.float32),
    )(x_seq, agg, deg, w_sr, w_neigh, b_diff, wm1, wm2, b_mix, wdt_row,
      b_dt, a_diag, b_mat, c_mat, dv, b_res)


def kernel(x_seq, edge_index_seq, W_self, W_neigh, b_diff, W_mix, b_mix,
           W_dt, b_dt, A_diag, B, C, Dv, W_res, b_res):
    ei = edge_index_seq.astype(jnp.int32)
    src = ei[:, 0, :]
    dst = ei[:, 1, :]
    pad = EPAD - E
    src_p = jnp.pad(src, ((0, 0), (0, pad)))
    dst_p = jnp.pad(dst, ((0, 0), (0, pad)), constant_values=N)
    xcols = (x_seq.reshape(L * N, NC, HC)
             .transpose(1, 0, 2).reshape(NC * L * N, HC))
    lofs = (jnp.arange(L, dtype=jnp.int32) * N)[None, :, None]
    cofs = (jnp.arange(NC, dtype=jnp.int32) * (L * N))[:, None, None]
    src_abs = src_p[None] + lofs + cofs
    srcq = src_abs.reshape(NC, L, NS, NCHUNK, CHUNK)
    dstq = dst_p.reshape(L, NS, NCHUNK, CHUNK)

    agg, deg = _sc_agg_call(xcols, srcq, dstq)

    w_sr = jnp.concatenate([W_self, W_res], axis=1)
    wm1 = W_mix[:HID]
    wm2 = W_mix[HID:]
    out = _tc_dense_call(
        x_seq, agg, deg, w_sr, W_neigh, b_diff.reshape(1, HID), wm1, wm2,
        b_mix.reshape(1, HID), W_dt.reshape(1, HID), b_dt.reshape(1, 1),
        A_diag.reshape(1, STATE), B, C, Dv.reshape(1, HID),
        b_res.reshape(1, HID))
    return out


# speedup vs baseline: 4.9878x; 1.0694x over previous
import jax
import jax.numpy as jnp
from jax import lax
from jax.experimental import pallas as pl
from jax.experimental.pallas import tpu as pltpu
from jax.experimental.pallas import tpu_sc as plsc

L, N, E = 4, 10000, 320000
IN_C, HID, STATE = 128, 256, 16

NC, NS = 2, 16
HC = IN_C // NC
CHUNK = 128
EPW = ((E + NS * 2 * CHUNK - 1) // (NS * 2 * CHUNK)) * (2 * CHUNK)
NCHUNK = EPW // CHUNK
NPAIR = NCHUNK // 2
EPAD = EPW * NS
NPAD = ((N + NS * 8 - 1) // (NS * 8)) * (NS * 8)
ROWS = NPAD // NS


def _sc_agg_body(xcols, srcq, dstq, zagg, zdeg, ones_src,
                 agg_out, deg_out,
                 src_v, dst_v, buf_a, buf_b, ones_v, agg_sh, deg_sh,
                 semg_a, semg_b, sems_a, sems_b, semo):
    cid = lax.axis_index("c")
    sid = lax.axis_index("s")

    pltpu.sync_copy(ones_src, ones_v)

    def gather(j, buf, sem):
        pltpu.async_copy(xcols.at[src_v.at[j]], buf, sem)

    def gather_wait(j, buf, sem):
        pltpu.make_async_copy(xcols.at[src_v.at[j]], buf, sem).wait()

    def scatter(j, buf, sem):
        pltpu.async_copy(buf, agg_sh.at[dst_v.at[j]], sem, add=True)

    def scatter_wait(j, buf, sem):
        pltpu.make_async_copy(buf, agg_sh.at[dst_v.at[j]], sem).wait()

    def deg_scatter(j):
        pltpu.async_copy(ones_v, deg_sh.at[dst_v.at[j]], semo, add=True)

    def deg_wait(j):
        pltpu.make_async_copy(ones_v, deg_sh.at[dst_v.at[j]], semo).wait()

    for l in range(L):
        pltpu.sync_copy(zagg.at[pl.ds(sid * ROWS, ROWS)],
                        agg_sh.at[pl.ds(sid * ROWS, ROWS)])
        pltpu.sync_copy(zdeg.at[pl.ds(sid * ROWS, ROWS)],
                        deg_sh.at[pl.ds(sid * ROWS, ROWS)])
        plsc.subcore_barrier()

        pltpu.sync_copy(srcq.at[cid, l, sid], src_v)
        pltpu.sync_copy(dstq.at[l, sid], dst_v)

        gather(0, buf_a, semg_a)

        def pair_body(i, carry):
            j0 = 2 * i
            j1 = 2 * i + 1

            @pl.when(i > 0)
            def _():
                scatter_wait(j1 - 2, buf_b, sems_b)
            gather_wait(j0, buf_a, semg_a)
            gather(j1, buf_b, semg_b)
            scatter(j0, buf_a, sems_a)

            @pl.when(i > 0)
            def _():
                deg_wait(j0 - 2 + cid)

            @pl.when(cid == 0)
            def _():
                deg_scatter(j0)
            gather_wait(j1, buf_b, semg_b)
            scatter_wait(j0, buf_a, sems_a)

            @pl.when(i < NPAIR - 1)
            def _():
                gather(j0 + 2, buf_a, semg_a)
            scatter(j1, buf_b, sems_b)

            @pl.when(cid == 1)
            def _():
                deg_scatter(j1)
            return carry

        lax.fori_loop(0, NPAIR, pair_body, 0)
        scatter_wait(NCHUNK - 1, buf_b, sems_b)
        deg_wait(NCHUNK - 2 + cid)
        plsc.subcore_barrier()

        pltpu.sync_copy(agg_sh.at[pl.ds(sid * ROWS, ROWS)],
                        agg_out.at[l, cid, pl.ds(sid * ROWS, ROWS)])
        pltpu.sync_copy(deg_sh.at[pl.ds(sid * ROWS, ROWS)],
                        deg_out.at[l, cid, pl.ds(sid * ROWS, ROWS)])
        plsc.subcore_barrier()


def _sc_agg_call(xcols, srcq, dstq):
    zagg = jnp.zeros((NPAD, HC), jnp.float32)
    zdeg = jnp.zeros((NPAD, 16), jnp.float32)
    ones_src = jnp.ones((CHUNK, 16), jnp.float32)
    return pl.kernel(
        _sc_agg_body,
        out_type=(
            jax.ShapeDtypeStruct((L, NC, NPAD, HC), jnp.float32),
            jax.ShapeDtypeStruct((L, NC, NPAD, 16), jnp.float32),
        ),
        mesh=plsc.VectorSubcoreMesh(core_axis_name="c", subcore_axis_name="s"),
        compiler_params=pltpu.CompilerParams(use_tc_tiling_on_sc=False),
        scratch_types=[
            pltpu.VMEM((NCHUNK, CHUNK), jnp.int32),
            pltpu.VMEM((NCHUNK, CHUNK), jnp.int32),
            pltpu.VMEM((CHUNK, HC), jnp.float32),
            pltpu.VMEM((CHUNK, HC), jnp.float32),
            pltpu.VMEM((CHUNK, 16), jnp.float32),
            pltpu.VMEM_SHARED((NPAD, HC), jnp.float32),
            pltpu.VMEM_SHARED((NPAD, 16), jnp.float32),
            pltpu.SemaphoreType.DMA,
            pltpu.SemaphoreType.DMA,
            pltpu.SemaphoreType.DMA,
            pltpu.SemaphoreType.DMA,
            pltpu.SemaphoreType.DMA,
        ],
    )(xcols, srcq, dstq, zagg, zdeg, ones_src)


NB = 1000


def _tc_dense_body(x_ref, agg_ref, deg_ref, wsr_ref, wn_ref, bdiff_ref,
                   wm1_ref, wm2_ref, bmix_ref, wdt_ref, bdt_ref, a_ref,
                   b_ref, c_ref, dv_ref, bres_ref, out_ref):
    s_a = jax.nn.softplus(a_ref[...])
    z_prev = None
    u_prev = None
    for l in range(L):
        x = x_ref[l]
        agg = jnp.concatenate([agg_ref[l, 0], agg_ref[l, 1]], axis=-1)
        deg = deg_ref[l, 0, :, 0:1] + deg_ref[l, 1, :, 0:1]
        aggn = agg / jnp.maximum(deg, 1.0)
        xsr = jnp.dot(x, wsr_ref[...], preferred_element_type=jnp.float32)
        z = (xsr[:, :HID]
             + jnp.dot(aggn, wn_ref[...], preferred_element_type=jnp.float32)
             + bdiff_ref[...])
        if l == 0:
            h = z
        else:
            gate = jax.nn.sigmoid(
                jnp.dot(z, wm1_ref[...], preferred_element_type=jnp.float32)
                + jnp.dot(z_prev, wm2_ref[...], preferred_element_type=jnp.float32)
                + bmix_ref[...])
            h = gate * z + (1.0 - gate) * z_prev
        dt = jax.nn.softplus(
            jnp.sum(h * wdt_ref[...], axis=1, keepdims=True) + bdt_ref[...])
        decay = jnp.exp(-dt * s_a)
        h_b = jnp.dot(h, b_ref[...], preferred_element_type=jnp.float32)
        if l == 0:
            u = h_b * dt
        else:
            u = u_prev * decay + h_b * dt
        y = (jnp.dot(u, c_ref[...], preferred_element_type=jnp.float32)
             + h * dv_ref[...])
        y_hat = y + xsr[:, HID:] + bres_ref[...]
        out_ref[l] = y_hat * jax.nn.sigmoid(y_hat)
        z_prev = z
        u_prev = u


def _tc_dense_call(x_seq, agg, deg, w_sr, w_neigh, b_diff, wm1, wm2, b_mix,
                   wdt_row, b_dt, a_diag, b_mat, c_mat, dv, b_res):
    grid = (N // NB,)
    full = lambda shape: pl.BlockSpec(shape, lambda i: (0,) * len(shape))
    return pl.pallas_call(
        _tc_dense_body,
        grid=grid,
        in_specs=[
            pl.BlockSpec((L, NB, IN_C), lambda i: (0, i, 0)),
            pl.BlockSpec((L, NC, NB, HC), lambda i: (0, 0, i, 0)),
            pl.BlockSpec((L, NC, NB, 16), lambda i: (0, 0, i, 0)),
            full((IN_C, 2 * HID)),
            full((IN_C, HID)),
            full((1, HID)),
            full((HID, HID)),
            full((HID, HID)),
            full((1, HID)),
            full((1, HID)),
            full((1, 1)),
            full((1, STATE)),
            full((HID, STATE)),
            full((STATE, HID)),
            full((1, HID)),
            full((1, HID)),
        ],
        out_specs=pl.BlockSpec((L, NB, HID), lambda i: (0, i, 0)),
        out_shape=jax.ShapeDtypeStruct((L, N, HID), jnp.float32),
    )(x_seq, agg, deg, w_sr, w_neigh, b_diff, wm1, wm2, b_mix, wdt_row,
      b_dt, a_diag, b_mat, c_mat, dv, b_res)


def kernel(x_seq, edge_index_seq, W_self, W_neigh, b_diff, W_mix, b_mix,
           W_dt, b_dt, A_diag, B, C, Dv, W_res, b_res):
    ei = edge_index_seq.astype(jnp.int32)
    src = ei[:, 0, :]
    dst = ei[:, 1, :]
    pad = EPAD - E
    src_p = jnp.pad(src, ((0, 0), (0, pad)))
    dst_p = jnp.pad(dst, ((0, 0), (0, pad)), constant_values=N)
    xcols = (x_seq.reshape(L * N, NC, HC)
             .transpose(1, 0, 2).reshape(NC * L * N, HC))
    lofs = (jnp.arange(L, dtype=jnp.int32) * N)[None, :, None]
    cofs = (jnp.arange(NC, dtype=jnp.int32) * (L * N))[:, None, None]
    src_abs = src_p[None] + lofs + cofs
    srcq = src_abs.reshape(NC, L, NS, NCHUNK, CHUNK)
    dstq = dst_p.reshape(L, NS, NCHUNK, CHUNK)

    agg, deg = _sc_agg_call(xcols, srcq, dstq)

    w_sr = jnp.concatenate([W_self, W_res], axis=1)
    wm1 = W_mix[:HID]
    wm2 = W_mix[HID:]
    out = _tc_dense_call(
        x_seq, agg, deg, w_sr, W_neigh, b_diff.reshape(1, HID), wm1, wm2,
        b_mix.reshape(1, HID), W_dt.reshape(1, HID), b_dt.reshape(1, 1),
        A_diag.reshape(1, STATE), B, C, Dv.reshape(1, HID),
        b_res.reshape(1, HID))
    return out


# speedup vs baseline: 6.3162x; 1.2663x over previous
import jax
import jax.numpy as jnp
from jax import lax
from jax.experimental import pallas as pl
from jax.experimental.pallas import tpu as pltpu
from jax.experimental.pallas import tpu_sc as plsc

L, N, E = 4, 10000, 320000
IN_C, HID, STATE = 128, 256, 16

NC, NS = 2, 16
HC = IN_C // NC
CHUNK = 128
EPW = ((E + NS * 2 * CHUNK - 1) // (NS * 2 * CHUNK)) * (2 * CHUNK)
NCHUNK = EPW // CHUNK
NPAIR = NCHUNK // 2
EPAD = EPW * NS
NPAD = ((N + NS * 8 - 1) // (NS * 8)) * (NS * 8)
ROWS = NPAD // NS


def _sc_agg_body(xcols, srcq, dstq, zagg, zdeg, ones_src,
                 agg_out, deg_out,
                 src_v, dst_v, buf_a, buf_b, ones_v, agg_sh, deg_sh,
                 semg_a, semg_b, sems_a, sems_b, semo):
    cid = lax.axis_index("c")
    sid = lax.axis_index("s")

    pltpu.sync_copy(ones_src, ones_v)

    def gather(j, buf, sem):
        pltpu.async_copy(xcols.at[src_v.at[j]], buf, sem)

    def gather_wait(j, buf, sem):
        pltpu.make_async_copy(xcols.at[src_v.at[j]], buf, sem).wait()

    def scatter(j, buf, sem):
        pltpu.async_copy(buf, agg_sh.at[dst_v.at[j]], sem, add=True)

    def scatter_wait(j, buf, sem):
        pltpu.make_async_copy(buf, agg_sh.at[dst_v.at[j]], sem).wait()

    def deg_scatter(j):
        pltpu.async_copy(ones_v, deg_sh.at[dst_v.at[j]], semo, add=True)

    def deg_wait(j):
        pltpu.make_async_copy(ones_v, deg_sh.at[dst_v.at[j]], semo).wait()

    for l in range(L):
        pltpu.sync_copy(zagg.at[pl.ds(sid * ROWS, ROWS)],
                        agg_sh.at[pl.ds(sid * ROWS, ROWS)])
        pltpu.sync_copy(zdeg.at[pl.ds(sid * ROWS, ROWS)],
                        deg_sh.at[pl.ds(sid * ROWS, ROWS)])
        plsc.subcore_barrier()

        pltpu.sync_copy(srcq.at[cid, l, sid], src_v)
        pltpu.sync_copy(dstq.at[l, sid], dst_v)

        gather(0, buf_a, semg_a)

        def pair_body(i, carry):
            j0 = 2 * i
            j1 = 2 * i + 1

            @pl.when(i > 0)
            def _():
                scatter_wait(j1 - 2, buf_b, sems_b)
            gather_wait(j0, buf_a, semg_a)
            gather(j1, buf_b, semg_b)
            scatter(j0, buf_a, sems_a)

            @pl.when(i > 0)
            def _():
                deg_wait(j0 - 2 + cid)

            @pl.when(cid == 0)
            def _():
                deg_scatter(j0)
            gather_wait(j1, buf_b, semg_b)
            scatter_wait(j0, buf_a, sems_a)

            @pl.when(i < NPAIR - 1)
            def _():
                gather(j0 + 2, buf_a, semg_a)
            scatter(j1, buf_b, sems_b)

            @pl.when(cid == 1)
            def _():
                deg_scatter(j1)
            return carry

        lax.fori_loop(0, NPAIR, pair_body, 0)
        scatter_wait(NCHUNK - 1, buf_b, sems_b)
        deg_wait(NCHUNK - 2 + cid)
        plsc.subcore_barrier()

        pltpu.sync_copy(agg_sh.at[pl.ds(sid * ROWS, ROWS)],
                        agg_out.at[l, cid, pl.ds(sid * ROWS, ROWS)])
        pltpu.sync_copy(deg_sh.at[pl.ds(sid * ROWS, ROWS)],
                        deg_out.at[l, cid, pl.ds(sid * ROWS, ROWS)])
        plsc.subcore_barrier()


def _sc_agg_call(xcols, srcq, dstq):
    zagg = jnp.zeros((NPAD, HC), jnp.bfloat16)
    zdeg = jnp.zeros((NPAD, 32), jnp.bfloat16)
    ones_src = jnp.ones((CHUNK, 32), jnp.bfloat16)
    return pl.kernel(
        _sc_agg_body,
        out_type=(
            jax.ShapeDtypeStruct((L, NC, NPAD, HC), jnp.bfloat16),
            jax.ShapeDtypeStruct((L, NC, NPAD, 32), jnp.bfloat16),
        ),
        mesh=plsc.VectorSubcoreMesh(core_axis_name="c", subcore_axis_name="s"),
        compiler_params=pltpu.CompilerParams(use_tc_tiling_on_sc=False),
        scratch_types=[
            pltpu.VMEM((NCHUNK, CHUNK), jnp.int32),
            pltpu.VMEM((NCHUNK, CHUNK), jnp.int32),
            pltpu.VMEM((CHUNK, HC), jnp.bfloat16),
            pltpu.VMEM((CHUNK, HC), jnp.bfloat16),
            pltpu.VMEM((CHUNK, 32), jnp.bfloat16),
            pltpu.VMEM_SHARED((NPAD, HC), jnp.bfloat16),
            pltpu.VMEM_SHARED((NPAD, 32), jnp.bfloat16),
            pltpu.SemaphoreType.DMA,
            pltpu.SemaphoreType.DMA,
            pltpu.SemaphoreType.DMA,
            pltpu.SemaphoreType.DMA,
            pltpu.SemaphoreType.DMA,
        ],
    )(xcols, srcq, dstq, zagg, zdeg, ones_src)


NB = 1000


def _tc_dense_body(x_ref, agg_ref, deg_ref, wsr_ref, wn_ref, bdiff_ref,
                   wm1_ref, wm2_ref, bmix_ref, wdt_ref, bdt_ref, a_ref,
                   b_ref, c_ref, dv_ref, bres_ref, out_ref):
    s_a = jax.nn.softplus(a_ref[...])
    z_prev = None
    u_prev = None
    for l in range(L):
        x = x_ref[l]
        agg = jnp.concatenate([agg_ref[l, 0], agg_ref[l, 1]],
                              axis=-1).astype(jnp.float32)
        deg = (deg_ref[l, 0, :, 0:1]
               + deg_ref[l, 1, :, 0:1]).astype(jnp.float32)
        aggn = agg / jnp.maximum(deg, 1.0)
        xsr = jnp.dot(x, wsr_ref[...], preferred_element_type=jnp.float32)
        z = (xsr[:, :HID]
             + jnp.dot(aggn, wn_ref[...], preferred_element_type=jnp.float32)
             + bdiff_ref[...])
        if l == 0:
            h = z
        else:
            gate = jax.nn.sigmoid(
                jnp.dot(z, wm1_ref[...], preferred_element_type=jnp.float32)
                + jnp.dot(z_prev, wm2_ref[...], preferred_element_type=jnp.float32)
                + bmix_ref[...])
            h = gate * z + (1.0 - gate) * z_prev
        dt = jax.nn.softplus(
            jnp.sum(h * wdt_ref[...], axis=1, keepdims=True) + bdt_ref[...])
        decay = jnp.exp(-dt * s_a)
        h_b = jnp.dot(h, b_ref[...], preferred_element_type=jnp.float32)
        if l == 0:
            u = h_b * dt
        else:
            u = u_prev * decay + h_b * dt
        y = (jnp.dot(u, c_ref[...], preferred_element_type=jnp.float32)
             + h * dv_ref[...])
        y_hat = y + xsr[:, HID:] + bres_ref[...]
        out_ref[l] = y_hat * jax.nn.sigmoid(y_hat)
        z_prev = z
        u_prev = u


def _tc_dense_call(x_seq, agg, deg, w_sr, w_neigh, b_diff, wm1, wm2, b_mix,
                   wdt_row, b_dt, a_diag, b_mat, c_mat, dv, b_res):
    grid = (N // NB,)
    full = lambda shape: pl.BlockSpec(shape, lambda i: (0,) * len(shape))
    return pl.pallas_call(
        _tc_dense_body,
        grid=grid,
        in_specs=[
            pl.BlockSpec((L, NB, IN_C), lambda i: (0, i, 0)),
            pl.BlockSpec((L, NC, NB, HC), lambda i: (0, 0, i, 0)),
            pl.BlockSpec((L, NC, NB, 32), lambda i: (0, 0, i, 0)),
            full((IN_C, 2 * HID)),
            full((IN_C, HID)),
            full((1, HID)),
            full((HID, HID)),
            full((HID, HID)),
            full((1, HID)),
            full((1, HID)),
            full((1, 1)),
            full((1, STATE)),
            full((HID, STATE)),
            full((STATE, HID)),
            full((1, HID)),
            full((1, HID)),
        ],
        out_specs=pl.BlockSpec((L, NB, HID), lambda i: (0, i, 0)),
        out_shape=jax.ShapeDtypeStruct((L, N, HID), jnp.float32),
    )(x_seq, agg, deg, w_sr, w_neigh, b_diff, wm1, wm2, b_mix, wdt_row,
      b_dt, a_diag, b_mat, c_mat, dv, b_res)


def kernel(x_seq, edge_index_seq, W_self, W_neigh, b_diff, W_mix, b_mix,
           W_dt, b_dt, A_diag, B, C, Dv, W_res, b_res):
    ei = edge_index_seq.astype(jnp.int32)
    src = ei[:, 0, :]
    dst = ei[:, 1, :]
    pad = EPAD - E
    src_p = jnp.pad(src, ((0, 0), (0, pad)))
    dst_p = jnp.pad(dst, ((0, 0), (0, pad)), constant_values=N)
    xcols = (x_seq.astype(jnp.bfloat16).reshape(L * N, NC, HC)
             .transpose(1, 0, 2).reshape(NC * L * N, HC))
    lofs = (jnp.arange(L, dtype=jnp.int32) * N)[None, :, None]
    cofs = (jnp.arange(NC, dtype=jnp.int32) * (L * N))[:, None, None]
    src_abs = src_p[None] + lofs + cofs
    srcq = src_abs.reshape(NC, L, NS, NCHUNK, CHUNK)
    dstq = dst_p.reshape(L, NS, NCHUNK, CHUNK)

    agg, deg = _sc_agg_call(xcols, srcq, dstq)

    w_sr = jnp.concatenate([W_self, W_res], axis=1)
    wm1 = W_mix[:HID]
    wm2 = W_mix[HID:]
    out = _tc_dense_call(
        x_seq, agg, deg, w_sr, W_neigh, b_diff.reshape(1, HID), wm1, wm2,
        b_mix.reshape(1, HID), W_dt.reshape(1, HID), b_dt.reshape(1, 1),
        A_diag.reshape(1, STATE), B, C, Dv.reshape(1, HID),
        b_res.reshape(1, HID))
    return out
